# baseline (speedup 1.0000x reference)
import jax
import jax.numpy as jnp
from jax import lax
from jax.experimental import pallas as pl
from jax.experimental.pallas import tpu as pltpu
from jax.experimental.pallas import tpu_sc as plsc

N, M, E = 50000, 10000, 800000
H, L = 64, 32

NC, NS = 2, 16

E_PAD = 819200
DW = 128

M_ACC = 10048
M_CNT = 10240
HALF = N // 2
H_ACC = 25600
H_CNT = 25600
NB = 2000
N_CNT = 64000

_f32 = jnp.float32
_i32 = jnp.int32


def _relu(x):
    return jnp.maximum(x, 0.0)


def _make_agg(acc_rows, cnt_rows, quota, remap, do_cnt, ndesc, dw):
    group = ndesc * dw
    n_groups = quota // group
    assert quota % group == 0 and dw % 8 == 0 and dw <= DW
    stripe = acc_rows // NS
    assert acc_rows % NS == 0
    if remap:
        bnd = HALF % stripe
        ch = next(c for c in range(min(stripe, group), 0, -1)
                  if stripe % c == 0 and bnd % c == 0)
    else:
        ch = next(c for c in range(min(stripe, group), 0, -1)
                  if stripe % c == 0)
    chunks = [(o, ch) for o in range(0, stripe, ch)]
    cnt_stripe = cnt_rows // NS if do_cnt else 0
    assert cnt_stripe % 16 == 0

    def body(*refs):
        if remap:
            (ps_hbm, pd_hbm, cnts_hbm, table_hbm), refs = refs[:4], refs[4:]
        else:
            (src_hbm, dst_hbm, table_hbm), refs = refs[:3], refs[3:]
        if do_cnt:
            (out_hbm, cnt_hbm), refs = refs[:2], refs[2:]
            (rows, idx_src, idx_dst1, idx_dst2, ones_v, cbounce,
             acc, cnt_acc), refs = refs[:8], refs[8:]
        else:
            (out_hbm,), refs = refs[:1], refs[1:]
            (rows, idx_src, idx_dst1, idx_dst2,
             acc), refs = refs[:5], refs[5:]
        if remap:
            (cnts_v,), refs = refs[:1], refs[1:]
        if do_cnt:
            isem, gsem, ssem, csem = refs
        else:
            isem, gsem, ssem = refs

        c = lax.axis_index("c")
        s = lax.axis_index("s")

        z16 = jnp.zeros((16,), _f32)

        def zrow(r, carry):
            for cc in range(H // 16):
                rows[0, r, pl.ds(cc * 16, 16)] = z16
            return carry

        lax.fori_loop(0, ch, zrow, 0)
        for (choff, chlen) in chunks:
            pltpu.sync_copy(rows.at[0, pl.ds(0, chlen)],
                            acc.at[pl.ds(s * stripe + choff, chlen)])
        if do_cnt:
            def zc(k, carry):
                cbounce[pl.ds(k * 16, 16)] = z16
                return carry
            lax.fori_loop(0, cnt_stripe // 16, zc, 0)
            pltpu.sync_copy(cbounce.at[pl.ds(0, cnt_stripe)],
                            cnt_acc.at[pl.ds(s * cnt_stripe, cnt_stripe)])
            one16 = jnp.ones((16,), _f32)

            def oc(k, carry):
                ones_v[pl.ds(k * 16, 16)] = one16
                return carry
            lax.fori_loop(0, dw // 16, oc, 0)
        plsc.subcore_barrier()

        if not remap:
            tile_base = (c * NS + s) * quota
        base_node = c * HALF

        def stage_idx(p, base):
            if remap:
                pltpu.async_copy(ps_hbm.at[c, pl.ds(base, group)],
                                 idx_src.at[p], isem.at[p])
                pltpu.async_copy(pd_hbm.at[c, pl.ds(base, group)],
                                 idx_dst1.at[p], isem.at[p])
            else:
                pltpu.async_copy(src_hbm.at[pl.ds(base, group)],
                                 idx_src.at[p], isem.at[p])
                pltpu.async_copy(dst_hbm.at[pl.ds(base, group)],
                                 idx_dst1.at[p], isem.at[p])

        def wait_idx_fire_gathers(p):
            if remap:
                srcref = ps_hbm.at[c, pl.ds(0, group)]
            else:
                srcref = src_hbm.at[pl.ds(0, group)]
            pltpu.make_async_copy(srcref, idx_src.at[p], isem.at[p]).wait()
            pltpu.make_async_copy(srcref, idx_dst1.at[p], isem.at[p]).wait()
            for j in range(ndesc):
                pltpu.async_copy(
                    table_hbm.at[idx_src.at[p, pl.ds(j * dw, dw)]],
                    rows.at[p, pl.ds(j * dw, dw)], gsem.at[p])

        def drain_gathers(p):
            for j in range(ndesc):
                pltpu.make_async_copy(
                    table_hbm.at[pl.ds(0, dw)],
                    rows.at[p, pl.ds(j * dw, dw)], gsem.at[p]).wait()

        def drain_scatters():
            for j in range(ndesc):
                pltpu.make_async_copy(
                    table_hbm.at[pl.ds(0, dw)],
                    rows.at[0, pl.ds(j * dw, dw)], ssem).wait()
            if do_cnt:
                for j in range(ndesc):
                    pltpu.make_async_copy(
                        cnt_hbm.at[pl.ds(0, dw)],
                        cbounce.at[pl.ds(0, dw)], csem).wait()

        def remap_fn(p):
            def rm(j, carry2):
                v = idx_dst1[p, pl.ds(j * 16, 16)]
                if remap:
                    local = v - base_node
                    oob = (local < 0) | (local >= HALF)
                    dump = HALF + (v & 63)
                    res = jnp.where(oob, dump, local)
                else:
                    res = v
                idx_dst2[p, j // (dw // 16),
                         pl.ds((j % (dw // 16)) * 16, 16)] = res
                return carry2
            lax.fori_loop(0, group // 16, rm, 0)

        def fire_scatters(p):
            for j in range(ndesc):
                pltpu.async_copy(rows.at[p, pl.ds(j * dw, dw)],
                                 acc.at[idx_dst2.at[p, j]], ssem, add=True)
            if do_cnt:
                for j in range(ndesc):
                    pltpu.async_copy(ones_v, cnt_acc.at[idx_dst2.at[p, j]],
                                     csem, add=True)

        if remap:
            for r_local in range(2):
                r = s * 2 + r_local
                pltpu.sync_copy(cnts_hbm.at[r], cnts_v)
                lane = lax.iota(_i32, 16)
                ng = jnp.sum(jnp.where(lane == c, cnts_v[pl.ds(0, 16)], 0))
                base_r = r * RCAP

                @pl.when(ng > 0)
                def _():
                    stage_idx(0, base_r)
                    wait_idx_fire_gathers(0)

                def group_body(g, carry):
                    p = g & 1

                    @pl.when(g >= 1)
                    def _():
                        drain_scatters()

                    @pl.when(g < ng - 1)
                    def _():
                        stage_idx(1 - p, base_r + (g + 1) * group)

                    remap_fn(p)
                    drain_gathers(p)

                    @pl.when(g < ng - 1)
                    def _():
                        wait_idx_fire_gathers(1 - p)

                    fire_scatters(p)
                    return carry

                lax.fori_loop(0, ng, group_body, 0)

                @pl.when(ng > 0)
                def _():
                    drain_scatters()
        else:
            stage_idx(0, tile_base)
            wait_idx_fire_gathers(0)

            def group_body(g, carry):
                p = g & 1

                @pl.when(g >= 1)
                def _():
                    drain_scatters()

                @pl.when(g < n_groups - 1)
                def _():
                    stage_idx(1 - p, tile_base + (g + 1) * group)

                remap_fn(p)
                drain_gathers(p)

                @pl.when(g < n_groups - 1)
                def _():
                    wait_idx_fire_gathers(1 - p)

                fire_scatters(p)
                return carry

            lax.fori_loop(0, n_groups, group_body, 0)
            drain_scatters()
        plsc.subcore_barrier()

        if remap:
            for (choff, chlen) in chunks:
                a0 = s * stripe + choff

                @pl.when(a0 < HALF)
                def _():
                    pltpu.sync_copy(acc.at[pl.ds(a0, chlen)],
                                    rows.at[0, pl.ds(0, chlen)])
                    pltpu.sync_copy(rows.at[0, pl.ds(0, chlen)],
                                    out_hbm.at[pl.ds(c * HALF + a0, chlen)])
            if do_cnt:
                chc = next(cc for cc in range(min(cnt_stripe, group), 0, -1)
                           if cnt_stripe % cc == 0 and (HALF % cnt_stripe) % cc == 0
                           and cc % 8 == 0)
                for co in range(0, cnt_stripe, chc):
                    c0 = s * cnt_stripe + co

                    @pl.when(c0 < HALF)
                    def _():
                        pltpu.sync_copy(cnt_acc.at[pl.ds(c0, chc)],
                                        cbounce.at[pl.ds(0, chc)])
                        pltpu.sync_copy(cbounce.at[pl.ds(0, chc)],
                                        cnt_hbm.at[pl.ds(c * HALF + c0, chc)])
        else:
            for (choff, chlen) in chunks:
                a0 = s * stripe + choff
                pltpu.sync_copy(acc.at[pl.ds(a0, chlen)],
                                rows.at[0, pl.ds(0, chlen)])
                pltpu.sync_copy(rows.at[0, pl.ds(0, chlen)],
                                out_hbm.at[pl.ds(c * acc_rows + a0, chlen)])
            if do_cnt:
                c0 = s * cnt_stripe
                pltpu.sync_copy(cnt_acc.at[pl.ds(c0, cnt_stripe)],
                                cbounce.at[pl.ds(0, cnt_stripe)])
                pltpu.sync_copy(cbounce.at[pl.ds(0, cnt_stripe)],
                                cnt_hbm.at[pl.ds(c * cnt_rows + c0, cnt_stripe)])

    mesh = plsc.VectorSubcoreMesh(core_axis_name="c", subcore_axis_name="s",
                                  num_cores=NC, num_subcores=NS)
    out_rows = NC * HALF if remap else NC * acc_rows
    out_type = [jax.ShapeDtypeStruct((out_rows, H), _f32)]
    scratch = [
        pltpu.VMEM((2, group, H), _f32),
        pltpu.VMEM((2, group), _i32),
        pltpu.VMEM((2, group), _i32),
        pltpu.VMEM((2, ndesc, dw), _i32),
    ]
    if do_cnt:
        cnt_out = NC * HALF if remap else NC * cnt_rows
        out_type.append(jax.ShapeDtypeStruct((cnt_out,), _f32))
        scratch += [pltpu.VMEM((dw,), _f32), pltpu.VMEM((cnt_stripe,), _f32)]
    scratch.append(pltpu.VMEM_SHARED((acc_rows, H), _f32))
    if do_cnt:
        scratch.append(pltpu.VMEM_SHARED((cnt_rows,), _f32))
    if remap:
        scratch.append(pltpu.VMEM((16,), _i32))
    scratch.append(pltpu.SemaphoreType.DMA((2,)))
    scratch.append(pltpu.SemaphoreType.DMA((2,)))
    scratch.append(pltpu.SemaphoreType.DMA)
    if do_cnt:
        scratch.append(pltpu.SemaphoreType.DMA)

    return pl.kernel(body, out_type=tuple(out_type), mesh=mesh,
                     scratch_types=tuple(scratch),
                     compiler_params=pltpu.CompilerParams(
                         use_tc_tiling_on_sc=False,
                         needs_layout_passes=False))


RCAP = 25760
PG = 160


def _make_partition():
    quota = E_PAD // (NC * NS)
    chunk = 1024
    nchunks = quota // chunk

    def body(src_hbm, dst_hbm, dn_hbm, ps_hbm, pd_hbm, cnts_hbm,
             cntn_hbm, cnte_hbm,
             sbuf, dbuf, nbuf, b0s, b0d, b1s, b1d, cvec,
             idx2n, idx2e, ones_v, cbounce, cntn_acc, cnte_acc, csem):
        c = lax.axis_index("c")
        s = lax.axis_index("s")
        r = c * NS + s
        tile_base = r * quota

        iota16 = lax.iota(_i32, 16)
        pdstv = (1 << 29) + lax.iota(_i32, 16)

        z16 = jnp.zeros((16,), _f32)
        nstripe = N_CNT // NS
        estripe = M_CNT // NS

        def zc(k, carry):
            cbounce[pl.ds(k * 16, 16)] = z16
            return carry
        lax.fori_loop(0, nstripe // 16, zc, 0)
        pltpu.sync_copy(cbounce.at[pl.ds(0, nstripe)],
                        cntn_acc.at[pl.ds(s * nstripe, nstripe)])
        pltpu.sync_copy(cbounce.at[pl.ds(0, estripe)],
                        cnte_acc.at[pl.ds(s * estripe, estripe)])
        one16 = jnp.ones((16,), _f32)

        def oc(k, carry):
            ones_v[pl.ds(k * 16, 16)] = one16
            return carry
        lax.fori_loop(0, 128 // 16, oc, 0)
        plsc.subcore_barrier()

        cur0, cur1 = 0, 0
        for ci in range(nchunks):
            base = tile_base + ci * chunk
            pltpu.sync_copy(src_hbm.at[pl.ds(base, chunk)], sbuf)
            pltpu.sync_copy(dst_hbm.at[pl.ds(base, chunk)], dbuf)
            pltpu.sync_copy(dn_hbm.at[pl.ds(base, chunk)], nbuf)

            def cstage(j, carry):
                vn = dbuf[pl.ds(j * 16, 16)]
                dmp = N + (vn & 63)
                vn = jnp.where(vn >= N, dmp, vn)
                idx2n[j // 8, pl.ds((j % 8) * 16, 16)] = vn
                ve = nbuf[pl.ds(j * 16, 16)]
                idx2e[j // 8, pl.ds((j % 8) * 16, 16)] = ve
                return carry
            lax.fori_loop(0, chunk // 16, cstage, 0)
            cds = [pltpu.async_copy(ones_v, cntn_acc.at[idx2n.at[j]],
                                    csem, add=True) for j in range(8)]
            cds += [pltpu.async_copy(ones_v, cnte_acc.at[idx2e.at[j]],
                                     csem, add=True) for j in range(8)]

            def step(j, carry):
                c0, c1 = carry
                vs = sbuf[pl.ds(j * 16, 16)]
                vd = dbuf[pl.ds(j * 16, 16)]
                m0 = vd < HALF
                m0i = m0.astype(_i32)
                n0 = jnp.sum(m0i)
                pos0 = c0 + plsc.cumsum(m0i) - 1
                plsc.store_scatter(b0s, [pos0], vs, mask=m0)
                plsc.store_scatter(b0d, [pos0], vd, mask=m0)
                m1 = jnp.logical_not(m0)
                pos1 = c1 + plsc.cumsum(1 - m0i) - 1
                plsc.store_scatter(b1s, [pos1], vs, mask=m1)
                plsc.store_scatter(b1d, [pos1], vd, mask=m1)
                return (c0 + n0, c1 + (16 - n0))

            cur0, cur1 = lax.fori_loop(0, chunk // 16, step, (cur0, cur1))
            for d in cds:
                d.wait()

        for k in range(PG // 16):
            b0s[pl.ds(cur0 + k * 16, 16)] = iota16
            b0d[pl.ds(cur0 + k * 16, 16)] = pdstv
            b1s[pl.ds(cur1 + k * 16, 16)] = iota16
            b1d[pl.ds(cur1 + k * 16, 16)] = pdstv

        ng0 = (cur0 + PG - 1) // PG
        ng1 = (cur1 + PG - 1) // PG
        cv = jnp.where(iota16 == 0, ng0, jnp.where(iota16 == 1, ng1, 0))
        cvec[pl.ds(0, 16)] = cv

        pltpu.sync_copy(b0s, ps_hbm.at[0, pl.ds(r * RCAP, RCAP)])
        pltpu.sync_copy(b0d, pd_hbm.at[0, pl.ds(r * RCAP, RCAP)])
        pltpu.sync_copy(b1s, ps_hbm.at[1, pl.ds(r * RCAP, RCAP)])
        pltpu.sync_copy(b1d, pd_hbm.at[1, pl.ds(r * RCAP, RCAP)])
        pltpu.sync_copy(cvec, cnts_hbm.at[r])
        plsc.subcore_barrier()
        pltpu.sync_copy(cntn_acc.at[pl.ds(s * nstripe, nstripe)],
                        cbounce.at[pl.ds(0, nstripe)])
        pltpu.sync_copy(cbounce.at[pl.ds(0, nstripe)],
                        cntn_hbm.at[pl.ds(c * N_CNT + s * nstripe, nstripe)])
        pltpu.sync_copy(cnte_acc.at[pl.ds(s * estripe, estripe)],
                        cbounce.at[pl.ds(0, estripe)])
        pltpu.sync_copy(cbounce.at[pl.ds(0, estripe)],
                        cnte_hbm.at[pl.ds(c * M_CNT + s * estripe, estripe)])

    mesh = plsc.VectorSubcoreMesh(core_axis_name="c", subcore_axis_name="s",
                                  num_cores=NC, num_subcores=NS)
    out_type = (
        jax.ShapeDtypeStruct((2, NC * NS * RCAP), _i32),
        jax.ShapeDtypeStruct((2, NC * NS * RCAP), _i32),
        jax.ShapeDtypeStruct((NC * NS, 16), _i32),
        jax.ShapeDtypeStruct((NC * N_CNT,), _f32),
        jax.ShapeDtypeStruct((NC * M_CNT,), _f32),
    )
    scratch = (
        pltpu.VMEM((1024,), _i32),
        pltpu.VMEM((1024,), _i32),
        pltpu.VMEM((1024,), _i32),
        pltpu.VMEM((RCAP,), _i32),
        pltpu.VMEM((RCAP,), _i32),
        pltpu.VMEM((RCAP,), _i32),
        pltpu.VMEM((RCAP,), _i32),
        pltpu.VMEM((16,), _i32),
        pltpu.VMEM((8, 128), _i32),
        pltpu.VMEM((8, 128), _i32),
        pltpu.VMEM((128,), _f32),
        pltpu.VMEM((N_CNT // NS,), _f32),
        pltpu.VMEM_SHARED((N_CNT,), _f32),
        pltpu.VMEM_SHARED((M_CNT,), _f32),
        pltpu.SemaphoreType.DMA,
    )
    return pl.kernel(body, out_type=out_type, mesh=mesh,
                     scratch_types=scratch,
                     compiler_params=pltpu.CompilerParams(
                         use_tc_tiling_on_sc=False,
                         needs_layout_passes=False))


def _prep_node_body(nx, npw, npb, hn_o):
    hn_o[...] = _relu(nx[...] * npw[...] + npb[...][None, :])


def _prep_edge_body(ex, epw, epb, he_o):
    he_o[...] = _relu(
        jnp.dot(ex[...], epw[...], preferred_element_type=_f32)
        + epb[...][None, :])


def _sage_rows(agg, cnt, xd, wl, bl, wr):
    mean = agg / jnp.maximum(cnt, 1.0)[:, None]
    return _relu(
        jnp.dot(mean, wl, preferred_element_type=_f32) + bl[None, :]
        + jnp.dot(xd, wr, preferred_element_type=_f32))


def _he_side(agg_e, cnt_e, he, wl, bl, wr, pool):
    def body(agg_r, cnt_r, he_r, wl_r, bl_r, wr_r, out_r):
        pe = agg_r[0:M, :] + agg_r[M_ACC:M_ACC + M, :]
        ce = cnt_r[0:M] + cnt_r[M_CNT:M_CNT + M]
        res = _sage_rows(pe, ce, he_r[...], wl_r[...], bl_r[...], wr_r[...])
        if pool:
            out_r[...] = jnp.sum(res, axis=0)[None, :]
        else:
            out_r[...] = res

    shape = (1, H) if pool else (M, H)
    return pl.pallas_call(
        body, out_shape=jax.ShapeDtypeStruct(shape, _f32),
    )(agg_e, cnt_e, he, wl, bl, wr)


def _hn_side(agg_n, cnt_n, hn, wl, bl, wr, pool):
    cnt_n3 = cnt_n.reshape(NC * N_CNT // NB, 1, NB)

    def body(agg_r, cnt_a, cnt_b, hn_r, wl_r, bl_r, wr_r, out_r):
        res = _sage_rows(agg_r[...], cnt_a[0, 0, :] + cnt_b[0, 0, :],
                         hn_r[...], wl_r[...], bl_r[...], wr_r[...])
        if pool:
            @pl.when(pl.program_id(0) == 0)
            def _():
                out_r[...] = jnp.zeros_like(out_r)
            out_r[...] += jnp.sum(res, axis=0)[None, :]
        else:
            out_r[...] = res

    grid = (N // NB,)
    in_specs = [
        pl.BlockSpec((NB, H), lambda i: (i, 0)),
        pl.BlockSpec((1, 1, NB), lambda i: (i, 0, 0)),
        pl.BlockSpec((1, 1, NB), lambda i: (N_CNT // NB + i, 0, 0)),
        pl.BlockSpec((NB, H), lambda i: (i, 0)),
        pl.BlockSpec((H, H), lambda i: (0, 0)),
        pl.BlockSpec((H,), lambda i: (0,)),
        pl.BlockSpec((H, H), lambda i: (0, 0)),
    ]
    if pool:
        out_specs = pl.BlockSpec((1, H), lambda i: (0, 0))
        out_shape = jax.ShapeDtypeStruct((1, H), _f32)
    else:
        out_specs = pl.BlockSpec((NB, H), lambda i: (i, 0))
        out_shape = jax.ShapeDtypeStruct((N, H), _f32)
    return pl.pallas_call(
        body, grid=grid, in_specs=in_specs, out_specs=out_specs,
        out_shape=out_shape,
    )(agg_n, cnt_n3, cnt_n3, hn, wl, bl, wr)


def _head_body(nsum, esum, ow, ob, out_o):
    pool = jnp.concatenate([nsum[0] / N, esum[0] / M])
    out_o[...] = (jnp.dot(pool[None, :], ow[...],
                          preferred_element_type=_f32)[0] + ob[...])


def kernel(node_x, edge_x, ei_n2e, ei_e2n,
           node_proj_w, node_proj_b, edge_proj_w, edge_proj_b,
           c1_n2e_wl, c1_n2e_bl, c1_n2e_wr, c1_e2n_wl, c1_e2n_bl, c1_e2n_wr,
           c2_n2e_wl, c2_n2e_bl, c2_n2e_wr, c2_e2n_wl, c2_e2n_bl, c2_e2n_wr,
           out_w, out_b):
    npad = E_PAD - E
    pad_src = (jnp.arange(npad, dtype=_i32) % 9973)
    pad_dst_m = M + (jnp.arange(npad, dtype=_i32) & 31)
    pad_dst_n = (1 << 29) + (jnp.arange(npad, dtype=_i32) & 63)

    src_n2e = jnp.concatenate([ei_n2e[0].astype(_i32), pad_src])
    dst_n2e = jnp.concatenate([ei_n2e[1].astype(_i32), pad_dst_m])
    src_e2n = jnp.concatenate([ei_e2n[0].astype(_i32), pad_src])
    dst_e2n = jnp.concatenate([ei_e2n[1].astype(_i32), pad_dst_n])

    nblk = 10
    hn = pl.pallas_call(
        _prep_node_body,
        grid=(nblk,),
        in_specs=[pl.BlockSpec((N // nblk, 1), lambda i: (i, 0)),
                  pl.BlockSpec((1, H), lambda i: (0, 0)),
                  pl.BlockSpec((H,), lambda i: (0,))],
        out_specs=pl.BlockSpec((N // nblk, H), lambda i: (i, 0)),
        out_shape=jax.ShapeDtypeStruct((N, H), _f32),
    )(node_x, node_proj_w, node_proj_b)
    he = pl.pallas_call(
        _prep_edge_body,
        out_shape=jax.ShapeDtypeStruct((M, H), _f32),
    )(edge_x, edge_proj_w, edge_proj_b)

    agg_m1 = _make_agg(M_ACC, 0, E_PAD // (NC * NS), False, False, 4, 128)
    agg_n1 = _make_agg(H_ACC, 0, E_PAD // NS, True, False, 2, 80)
    agg_m2 = _make_agg(M_ACC, 0, E_PAD // (NC * NS), False, False, 4, 128)
    agg_n2 = _make_agg(H_ACC, 0, E_PAD // NS, True, False, 2, 80)

    plist_src, plist_dst, pcounts, cnt_n, cnt_e = _make_partition()(
        src_e2n, dst_e2n, dst_n2e)

    (agg_e1,) = agg_m1(src_n2e, dst_n2e, hn)
    (agg_nn1,) = agg_n1(plist_src, plist_dst, pcounts, he)

    he1 = _he_side(agg_e1, cnt_e, he, c1_n2e_wl, c1_n2e_bl, c1_n2e_wr,
                   pool=False)
    hn1 = _hn_side(agg_nn1, cnt_n, hn, c1_e2n_wl, c1_e2n_bl, c1_e2n_wr,
                   pool=False)

    (agg_e2,) = agg_m2(src_n2e, dst_n2e, hn1)
    (agg_nn2,) = agg_n2(plist_src, plist_dst, pcounts, he1)

    esum = _he_side(agg_e2, cnt_e, he1, c2_n2e_wl, c2_n2e_bl, c2_n2e_wr,
                    pool=True)
    nsum = _hn_side(agg_nn2, cnt_n, hn1, c2_e2n_wl, c2_e2n_bl, c2_e2n_wr,
                    pool=True)

    out = pl.pallas_call(
        _head_body,
        out_shape=jax.ShapeDtypeStruct((L,), _f32),
    )(nsum, esum, out_w, out_b)
    return out


# speedup vs baseline: 1.0086x; 1.0086x over previous
import jax
import jax.numpy as jnp
from jax import lax
from jax.experimental import pallas as pl
from jax.experimental.pallas import tpu as pltpu
from jax.experimental.pallas import tpu_sc as plsc

N, M, E = 50000, 10000, 800000
H, L = 64, 32

NC, NS = 2, 16

E_PAD = 819200
DW = 128

M_ACC = 10048
M_CNT = 10240
HALF = N // 2
H_ACC = 25600
H_CNT = 25600
NB = 5000

_f32 = jnp.float32
_i32 = jnp.int32


def _relu(x):
    return jnp.maximum(x, 0.0)


def _make_agg(acc_rows, cnt_rows, quota, remap, do_cnt, ndesc, dw):
    group = ndesc * dw
    n_groups = quota // group
    assert quota % group == 0 and dw % 8 == 0 and dw <= DW
    stripe = acc_rows // NS
    assert acc_rows % NS == 0
    if remap:
        bnd = HALF % stripe
        ch = next(c for c in range(min(stripe, group), 0, -1)
                  if stripe % c == 0 and bnd % c == 0)
    else:
        ch = next(c for c in range(min(stripe, group), 0, -1)
                  if stripe % c == 0)
    chunks = [(o, ch) for o in range(0, stripe, ch)]
    cnt_stripe = cnt_rows // NS if do_cnt else 0
    assert cnt_stripe % 16 == 0

    def body(*refs):
        if remap:
            (ps_hbm, pd_hbm, cnts_hbm, table_hbm), refs = refs[:4], refs[4:]
        else:
            (src_hbm, dst_hbm, table_hbm), refs = refs[:3], refs[3:]
        if do_cnt:
            (out_hbm, cnt_hbm), refs = refs[:2], refs[2:]
            (rows, idx_src, idx_dst1, idx_dst2, ones_v, cbounce,
             acc, cnt_acc), refs = refs[:8], refs[8:]
        else:
            (out_hbm,), refs = refs[:1], refs[1:]
            (rows, idx_src, idx_dst1, idx_dst2,
             acc), refs = refs[:5], refs[5:]
        if remap:
            (cnts_v,), refs = refs[:1], refs[1:]
        if do_cnt:
            isem, gsem, ssem, csem = refs
        else:
            isem, gsem, ssem = refs

        c = lax.axis_index("c")
        s = lax.axis_index("s")

        z16 = jnp.zeros((16,), _f32)

        def zrow(r, carry):
            for cc in range(H // 16):
                rows[0, r, pl.ds(cc * 16, 16)] = z16
            return carry

        lax.fori_loop(0, ch, zrow, 0)
        for (choff, chlen) in chunks:
            pltpu.sync_copy(rows.at[0, pl.ds(0, chlen)],
                            acc.at[pl.ds(s * stripe + choff, chlen)])
        if do_cnt:
            def zc(k, carry):
                cbounce[pl.ds(k * 16, 16)] = z16
                return carry
            lax.fori_loop(0, cnt_stripe // 16, zc, 0)
            pltpu.sync_copy(cbounce.at[pl.ds(0, cnt_stripe)],
                            cnt_acc.at[pl.ds(s * cnt_stripe, cnt_stripe)])
            one16 = jnp.ones((16,), _f32)

            def oc(k, carry):
                ones_v[pl.ds(k * 16, 16)] = one16
                return carry
            lax.fori_loop(0, dw // 16, oc, 0)
        plsc.subcore_barrier()

        if not remap:
            tile_base = (c * NS + s) * quota
        base_node = c * HALF

        def stage_idx(p, base):
            if remap:
                pltpu.async_copy(ps_hbm.at[c, pl.ds(base, group)],
                                 idx_src.at[p], isem.at[p])
                pltpu.async_copy(pd_hbm.at[c, pl.ds(base, group)],
                                 idx_dst1.at[p], isem.at[p])
            else:
                pltpu.async_copy(src_hbm.at[pl.ds(base, group)],
                                 idx_src.at[p], isem.at[p])
                pltpu.async_copy(dst_hbm.at[pl.ds(base, group)],
                                 idx_dst1.at[p], isem.at[p])

        def wait_idx_fire_gathers(p):
            if remap:
                srcref = ps_hbm.at[c, pl.ds(0, group)]
            else:
                srcref = src_hbm.at[pl.ds(0, group)]
            pltpu.make_async_copy(srcref, idx_src.at[p], isem.at[p]).wait()
            pltpu.make_async_copy(srcref, idx_dst1.at[p], isem.at[p]).wait()
            for j in range(ndesc):
                pltpu.async_copy(
                    table_hbm.at[idx_src.at[p, pl.ds(j * dw, dw)]],
                    rows.at[p, pl.ds(j * dw, dw)], gsem.at[p])

        def drain_gathers(p):
            for j in range(ndesc):
                pltpu.make_async_copy(
                    table_hbm.at[pl.ds(0, dw)],
                    rows.at[p, pl.ds(j * dw, dw)], gsem.at[p]).wait()

        def drain_scatters():
            for j in range(ndesc):
                pltpu.make_async_copy(
                    table_hbm.at[pl.ds(0, dw)],
                    rows.at[0, pl.ds(j * dw, dw)], ssem).wait()
            if do_cnt:
                for j in range(ndesc):
                    pltpu.make_async_copy(
                        cnt_hbm.at[pl.ds(0, dw)],
                        cbounce.at[pl.ds(0, dw)], csem).wait()

        def remap_fn(p):
            def rm(j, carry2):
                v = idx_dst1[p, pl.ds(j * 16, 16)]
                if remap:
                    local = v - base_node
                    oob = (local < 0) | (local >= HALF)
                    dump = HALF + (v & 63)
                    res = jnp.where(oob, dump, local)
                else:
                    res = v
                idx_dst2[p, j // (dw // 16),
                         pl.ds((j % (dw // 16)) * 16, 16)] = res
                return carry2
            lax.fori_loop(0, group // 16, rm, 0)

        def fire_scatters(p):
            for j in range(ndesc):
                pltpu.async_copy(rows.at[p, pl.ds(j * dw, dw)],
                                 acc.at[idx_dst2.at[p, j]], ssem, add=True)
            if do_cnt:
                for j in range(ndesc):
                    pltpu.async_copy(ones_v, cnt_acc.at[idx_dst2.at[p, j]],
                                     csem, add=True)

        if remap:
            for r_local in range(2):
                r = s * 2 + r_local
                pltpu.sync_copy(cnts_hbm.at[r], cnts_v)
                lane = lax.iota(_i32, 16)
                ng = jnp.sum(jnp.where(lane == c, cnts_v[pl.ds(0, 16)], 0))
                base_r = r * RCAP

                @pl.when(ng > 0)
                def _():
                    stage_idx(0, base_r)
                    wait_idx_fire_gathers(0)

                def group_body(g, carry):
                    p = g & 1

                    @pl.when(g >= 1)
                    def _():
                        drain_scatters()

                    @pl.when(g < ng - 1)
                    def _():
                        stage_idx(1 - p, base_r + (g + 1) * group)

                    remap_fn(p)
                    drain_gathers(p)

                    @pl.when(g < ng - 1)
                    def _():
                        wait_idx_fire_gathers(1 - p)

                    fire_scatters(p)
                    return carry

                lax.fori_loop(0, ng, group_body, 0)

                @pl.when(ng > 0)
                def _():
                    drain_scatters()
        else:
            stage_idx(0, tile_base)
            wait_idx_fire_gathers(0)

            def group_body(g, carry):
                p = g & 1

                @pl.when(g >= 1)
                def _():
                    drain_scatters()

                @pl.when(g < n_groups - 1)
                def _():
                    stage_idx(1 - p, tile_base + (g + 1) * group)

                remap_fn(p)
                drain_gathers(p)

                @pl.when(g < n_groups - 1)
                def _():
                    wait_idx_fire_gathers(1 - p)

                fire_scatters(p)
                return carry

            lax.fori_loop(0, n_groups, group_body, 0)
            drain_scatters()
        plsc.subcore_barrier()

        if remap:
            for (choff, chlen) in chunks:
                a0 = s * stripe + choff

                @pl.when(a0 < HALF)
                def _():
                    pltpu.sync_copy(acc.at[pl.ds(a0, chlen)],
                                    rows.at[0, pl.ds(0, chlen)])
                    pltpu.sync_copy(rows.at[0, pl.ds(0, chlen)],
                                    out_hbm.at[pl.ds(c * HALF + a0, chlen)])
            if do_cnt:
                chc = next(cc for cc in range(min(cnt_stripe, group), 0, -1)
                           if cnt_stripe % cc == 0 and (HALF % cnt_stripe) % cc == 0
                           and cc % 8 == 0)
                for co in range(0, cnt_stripe, chc):
                    c0 = s * cnt_stripe + co

                    @pl.when(c0 < HALF)
                    def _():
                        pltpu.sync_copy(cnt_acc.at[pl.ds(c0, chc)],
                                        cbounce.at[pl.ds(0, chc)])
                        pltpu.sync_copy(cbounce.at[pl.ds(0, chc)],
                                        cnt_hbm.at[pl.ds(c * HALF + c0, chc)])
        else:
            for (choff, chlen) in chunks:
                a0 = s * stripe + choff
                pltpu.sync_copy(acc.at[pl.ds(a0, chlen)],
                                rows.at[0, pl.ds(0, chlen)])
                pltpu.sync_copy(rows.at[0, pl.ds(0, chlen)],
                                out_hbm.at[pl.ds(c * acc_rows + a0, chlen)])
            if do_cnt:
                c0 = s * cnt_stripe
                pltpu.sync_copy(cnt_acc.at[pl.ds(c0, cnt_stripe)],
                                cbounce.at[pl.ds(0, cnt_stripe)])
                pltpu.sync_copy(cbounce.at[pl.ds(0, cnt_stripe)],
                                cnt_hbm.at[pl.ds(c * cnt_rows + c0, cnt_stripe)])

    mesh = plsc.VectorSubcoreMesh(core_axis_name="c", subcore_axis_name="s",
                                  num_cores=NC, num_subcores=NS)
    out_rows = NC * HALF if remap else NC * acc_rows
    out_type = [jax.ShapeDtypeStruct((out_rows, H), _f32)]
    scratch = [
        pltpu.VMEM((2, group, H), _f32),
        pltpu.VMEM((2, group), _i32),
        pltpu.VMEM((2, group), _i32),
        pltpu.VMEM((2, ndesc, dw), _i32),
    ]
    if do_cnt:
        cnt_out = NC * HALF if remap else NC * cnt_rows
        out_type.append(jax.ShapeDtypeStruct((cnt_out,), _f32))
        scratch += [pltpu.VMEM((dw,), _f32), pltpu.VMEM((cnt_stripe,), _f32)]
    scratch.append(pltpu.VMEM_SHARED((acc_rows, H), _f32))
    if do_cnt:
        scratch.append(pltpu.VMEM_SHARED((cnt_rows,), _f32))
    if remap:
        scratch.append(pltpu.VMEM((16,), _i32))
    scratch.append(pltpu.SemaphoreType.DMA((2,)))
    scratch.append(pltpu.SemaphoreType.DMA((2,)))
    scratch.append(pltpu.SemaphoreType.DMA)
    if do_cnt:
        scratch.append(pltpu.SemaphoreType.DMA)

    return pl.kernel(body, out_type=tuple(out_type), mesh=mesh,
                     scratch_types=tuple(scratch),
                     compiler_params=pltpu.CompilerParams(
                         use_tc_tiling_on_sc=False,
                         needs_layout_passes=False))


RCAP = 25760
PG = 160


def _make_partition():
    quota = E_PAD // (NC * NS)
    chunk = 1024
    nchunks = quota // chunk

    def body(src_hbm, dst_hbm, ps_hbm, pd_hbm, cnts_hbm,
             sbuf, dbuf, b0s, b0d, b1s, b1d, cvec):
        c = lax.axis_index("c")
        s = lax.axis_index("s")
        r = c * NS + s
        tile_base = r * quota

        iota16 = lax.iota(_i32, 16)
        pdstv = (1 << 29) + lax.iota(_i32, 16)

        cur0, cur1 = 0, 0
        for ci in range(nchunks):
            base = tile_base + ci * chunk
            pltpu.sync_copy(src_hbm.at[pl.ds(base, chunk)], sbuf)
            pltpu.sync_copy(dst_hbm.at[pl.ds(base, chunk)], dbuf)

            def step(j, carry):
                c0, c1 = carry
                vs = sbuf[pl.ds(j * 16, 16)]
                vd = dbuf[pl.ds(j * 16, 16)]
                m0 = vd < HALF
                m0i = m0.astype(_i32)
                n0 = jnp.sum(m0i)
                pos0 = c0 + plsc.cumsum(m0i) - 1
                plsc.store_scatter(b0s, [pos0], vs, mask=m0)
                plsc.store_scatter(b0d, [pos0], vd, mask=m0)
                m1 = jnp.logical_not(m0)
                pos1 = c1 + plsc.cumsum(1 - m0i) - 1
                plsc.store_scatter(b1s, [pos1], vs, mask=m1)
                plsc.store_scatter(b1d, [pos1], vd, mask=m1)
                return (c0 + n0, c1 + (16 - n0))

            cur0, cur1 = lax.fori_loop(0, chunk // 16, step, (cur0, cur1))

        for k in range(PG // 16):
            b0s[pl.ds(cur0 + k * 16, 16)] = iota16
            b0d[pl.ds(cur0 + k * 16, 16)] = pdstv
            b1s[pl.ds(cur1 + k * 16, 16)] = iota16
            b1d[pl.ds(cur1 + k * 16, 16)] = pdstv

        ng0 = (cur0 + PG - 1) // PG
        ng1 = (cur1 + PG - 1) // PG
        cv = jnp.where(iota16 == 0, ng0, jnp.where(iota16 == 1, ng1, 0))
        cvec[pl.ds(0, 16)] = cv

        pltpu.sync_copy(b0s, ps_hbm.at[0, pl.ds(r * RCAP, RCAP)])
        pltpu.sync_copy(b0d, pd_hbm.at[0, pl.ds(r * RCAP, RCAP)])
        pltpu.sync_copy(b1s, ps_hbm.at[1, pl.ds(r * RCAP, RCAP)])
        pltpu.sync_copy(b1d, pd_hbm.at[1, pl.ds(r * RCAP, RCAP)])
        pltpu.sync_copy(cvec, cnts_hbm.at[r])

    mesh = plsc.VectorSubcoreMesh(core_axis_name="c", subcore_axis_name="s",
                                  num_cores=NC, num_subcores=NS)
    out_type = (
        jax.ShapeDtypeStruct((2, NC * NS * RCAP), _i32),
        jax.ShapeDtypeStruct((2, NC * NS * RCAP), _i32),
        jax.ShapeDtypeStruct((NC * NS, 16), _i32),
    )
    scratch = (
        pltpu.VMEM((1024,), _i32),
        pltpu.VMEM((1024,), _i32),
        pltpu.VMEM((RCAP,), _i32),
        pltpu.VMEM((RCAP,), _i32),
        pltpu.VMEM((RCAP,), _i32),
        pltpu.VMEM((RCAP,), _i32),
        pltpu.VMEM((16,), _i32),
    )
    return pl.kernel(body, out_type=out_type, mesh=mesh,
                     scratch_types=scratch,
                     compiler_params=pltpu.CompilerParams(
                         use_tc_tiling_on_sc=False,
                         needs_layout_passes=False))


def _prep_node_body(nx, npw, npb, hn_o):
    hn_o[...] = _relu(nx[...] * npw[...] + npb[...][None, :])


def _prep_edge_body(ex, epw, epb, he_o):
    he_o[...] = _relu(
        jnp.dot(ex[...], epw[...], preferred_element_type=_f32)
        + epb[...][None, :])


def _sage_rows(agg, cnt, xd, wl, bl, wr):
    mean = agg / jnp.maximum(cnt, 1.0)[:, None]
    return _relu(
        jnp.dot(mean, wl, preferred_element_type=_f32) + bl[None, :]
        + jnp.dot(xd, wr, preferred_element_type=_f32))


def _he_side(agg_e, cnt_e, he, wl, bl, wr, pool):
    def body(agg_r, cnt_r, he_r, wl_r, bl_r, wr_r, out_r):
        pe = agg_r[0:M, :] + agg_r[M_ACC:M_ACC + M, :]
        ce = cnt_r[0:M] + cnt_r[M_CNT:M_CNT + M]
        res = _sage_rows(pe, ce, he_r[...], wl_r[...], bl_r[...], wr_r[...])
        if pool:
            out_r[...] = jnp.sum(res, axis=0)[None, :]
        else:
            out_r[...] = res

    shape = (1, H) if pool else (M, H)
    return pl.pallas_call(
        body, out_shape=jax.ShapeDtypeStruct(shape, _f32),
    )(agg_e, cnt_e, he, wl, bl, wr)


def _hn_side(agg_n, cnt_n, hn, wl, bl, wr, pool):
    cnt_n3 = cnt_n.reshape(N // NB, 1, NB)

    def body(agg_r, cnt_r, hn_r, wl_r, bl_r, wr_r, out_r):
        res = _sage_rows(agg_r[...], cnt_r[0, 0, :], hn_r[...],
                         wl_r[...], bl_r[...], wr_r[...])
        if pool:
            @pl.when(pl.program_id(0) == 0)
            def _():
                out_r[...] = jnp.zeros_like(out_r)
            out_r[...] += jnp.sum(res, axis=0)[None, :]
        else:
            out_r[...] = res

    grid = (N // NB,)
    in_specs = [
        pl.BlockSpec((NB, H), lambda i: (i, 0)),
        pl.BlockSpec((1, 1, NB), lambda i: (i, 0, 0)),
        pl.BlockSpec((NB, H), lambda i: (i, 0)),
        pl.BlockSpec((H, H), lambda i: (0, 0)),
        pl.BlockSpec((H,), lambda i: (0,)),
        pl.BlockSpec((H, H), lambda i: (0, 0)),
    ]
    if pool:
        out_specs = pl.BlockSpec((1, H), lambda i: (0, 0))
        out_shape = jax.ShapeDtypeStruct((1, H), _f32)
    else:
        out_specs = pl.BlockSpec((NB, H), lambda i: (i, 0))
        out_shape = jax.ShapeDtypeStruct((N, H), _f32)
    return pl.pallas_call(
        body, grid=grid, in_specs=in_specs, out_specs=out_specs,
        out_shape=out_shape,
    )(agg_n, cnt_n3, hn, wl, bl, wr)


def _head_body(nsum, esum, ow, ob, out_o):
    pool = jnp.concatenate([nsum[0] / N, esum[0] / M])
    out_o[...] = (jnp.dot(pool[None, :], ow[...],
                          preferred_element_type=_f32)[0] + ob[...])


def kernel(node_x, edge_x, ei_n2e, ei_e2n,
           node_proj_w, node_proj_b, edge_proj_w, edge_proj_b,
           c1_n2e_wl, c1_n2e_bl, c1_n2e_wr, c1_e2n_wl, c1_e2n_bl, c1_e2n_wr,
           c2_n2e_wl, c2_n2e_bl, c2_n2e_wr, c2_e2n_wl, c2_e2n_bl, c2_e2n_wr,
           out_w, out_b):
    npad = E_PAD - E
    pad_src = (jnp.arange(npad, dtype=_i32) % 9973)
    pad_dst_m = M + (jnp.arange(npad, dtype=_i32) & 31)
    pad_dst_n = (1 << 29) + (jnp.arange(npad, dtype=_i32) & 63)

    src_n2e = jnp.concatenate([ei_n2e[0].astype(_i32), pad_src])
    dst_n2e = jnp.concatenate([ei_n2e[1].astype(_i32), pad_dst_m])
    src_e2n = jnp.concatenate([ei_e2n[0].astype(_i32), pad_src])
    dst_e2n = jnp.concatenate([ei_e2n[1].astype(_i32), pad_dst_n])

    nblk = 10
    hn = pl.pallas_call(
        _prep_node_body,
        grid=(nblk,),
        in_specs=[pl.BlockSpec((N // nblk, 1), lambda i: (i, 0)),
                  pl.BlockSpec((1, H), lambda i: (0, 0)),
                  pl.BlockSpec((H,), lambda i: (0,))],
        out_specs=pl.BlockSpec((N // nblk, H), lambda i: (i, 0)),
        out_shape=jax.ShapeDtypeStruct((N, H), _f32),
    )(node_x, node_proj_w, node_proj_b)
    he = pl.pallas_call(
        _prep_edge_body,
        out_shape=jax.ShapeDtypeStruct((M, H), _f32),
    )(edge_x, edge_proj_w, edge_proj_b)

    agg_m1 = _make_agg(M_ACC, M_CNT, E_PAD // (NC * NS), False, True, 5, 128)
    agg_n1 = _make_agg(H_ACC, H_CNT, E_PAD // NS, True, True, 2, 80)
    agg_m2 = _make_agg(M_ACC, 0, E_PAD // (NC * NS), False, False, 5, 128)
    agg_n2 = _make_agg(H_ACC, 0, E_PAD // NS, True, False, 2, 80)

    plist_src, plist_dst, pcounts = _make_partition()(src_e2n, dst_e2n)

    agg_e1, cnt_e = agg_m1(src_n2e, dst_n2e, hn)
    agg_nn1, cnt_n = agg_n1(plist_src, plist_dst, pcounts, he)

    he1 = _he_side(agg_e1, cnt_e, he, c1_n2e_wl, c1_n2e_bl, c1_n2e_wr,
                   pool=False)
    hn1 = _hn_side(agg_nn1, cnt_n, hn, c1_e2n_wl, c1_e2n_bl, c1_e2n_wr,
                   pool=False)

    (agg_e2,) = agg_m2(src_n2e, dst_n2e, hn1)
    (agg_nn2,) = agg_n2(plist_src, plist_dst, pcounts, he1)

    esum = _he_side(agg_e2, cnt_e, he1, c2_n2e_wl, c2_n2e_bl, c2_n2e_wr,
                    pool=True)
    nsum = _hn_side(agg_nn2, cnt_n, hn1, c2_e2n_wl, c2_e2n_bl, c2_e2n_wr,
                    pool=True)

    out = pl.pallas_call(
        _head_body,
        out_shape=jax.ShapeDtypeStruct((L,), _f32),
    )(nsum, esum, out_w, out_b)
    return out
